# Initial kernel scaffold; baseline (speedup 1.0000x reference)
import jax
import jax.numpy as jnp
from jax.experimental import pallas as pl


def kernel(hidden_states, gate_w, w1, w2, w3):
    raise NotImplementedError("write your pallas kernel here")


# speedup vs baseline: 1.2722x; 1.2722x over previous
import functools

import jax
import jax.numpy as jnp
from jax import lax
from jax.experimental import pallas as pl
from jax.experimental.pallas import tpu as pltpu

NUM_EXPERTS = 8
TOP_K = 2
HIDDEN = 2048
FFN = 5632
TOKENS = 2048
F_CHUNK = 256
N_F = FFN // F_CHUNK


def _router_body(x_ref, gate_ref, p_ref):
    x = x_ref[...]
    gate = gate_ref[...]
    logits = jax.lax.dot_general(
        x, gate, (((1,), (1,)), ((), ())),
        preferred_element_type=jnp.float32)
    e_iota = lax.broadcasted_iota(jnp.int32, (TOKENS, NUM_EXPERTS), 1)
    big = jnp.float32(1e30)
    m1 = jnp.max(logits, axis=1, keepdims=True)
    i1 = jnp.min(jnp.where(logits >= m1, e_iota, NUM_EXPERTS), axis=1,
                 keepdims=True)
    oh1 = (e_iota == i1)
    logits2 = jnp.where(oh1, -big, logits)
    m2 = jnp.max(logits2, axis=1, keepdims=True)
    i2 = jnp.min(jnp.where(logits2 >= m2, e_iota, NUM_EXPERTS), axis=1,
                 keepdims=True)
    oh2 = (e_iota == i2)
    w1 = 1.0 / (1.0 + jnp.exp(m2 - m1))
    w2 = 1.0 - w1
    p_ref[...] = jnp.where(oh1, w1, 0.0) + jnp.where(oh2, w2, 0.0)


def _expert_body(x_ref, p_ref, w1_ref, w3_ref, w2_ref, out_ref):
    e = pl.program_id(0)
    f = pl.program_id(1)

    @pl.when((e == 0) & (f == 0))
    def _init():
        out_ref[...] = jnp.zeros_like(out_ref)

    x = x_ref[...].astype(jnp.bfloat16)
    w1 = w1_ref[0].astype(jnp.bfloat16)
    w3 = w3_ref[0].astype(jnp.bfloat16)
    w2 = w2_ref[0].astype(jnp.bfloat16)
    t1 = jax.lax.dot_general(x, w1, (((1,), (1,)), ((), ())),
                             preferred_element_type=jnp.float32)
    t3 = jax.lax.dot_general(x, w3, (((1,), (1,)), ((), ())),
                             preferred_element_type=jnp.float32)
    h = (t1 * jax.nn.sigmoid(t1) * t3).astype(jnp.bfloat16)
    cur = jax.lax.dot_general(h, w2, (((1,), (1,)), ((), ())),
                              preferred_element_type=jnp.float32)
    e_iota = lax.broadcasted_iota(jnp.int32, (TOKENS, NUM_EXPERTS), 1)
    pe = jnp.sum(jnp.where(e_iota == e, p_ref[...], 0.0), axis=1,
                 keepdims=True)
    out_ref[...] += pe * cur


@jax.jit
def _moe(x, gate_w, w1, w2, w3):
    p = pl.pallas_call(
        _router_body,
        out_shape=jax.ShapeDtypeStruct((TOKENS, NUM_EXPERTS), jnp.float32),
    )(x, gate_w)

    out = pl.pallas_call(
        _expert_body,
        grid=(NUM_EXPERTS, N_F),
        in_specs=[
            pl.BlockSpec((TOKENS, HIDDEN), lambda e, f: (0, 0)),
            pl.BlockSpec((TOKENS, NUM_EXPERTS), lambda e, f: (0, 0)),
            pl.BlockSpec((1, F_CHUNK, HIDDEN), lambda e, f: (e, f, 0)),
            pl.BlockSpec((1, F_CHUNK, HIDDEN), lambda e, f: (e, f, 0)),
            pl.BlockSpec((1, HIDDEN, F_CHUNK), lambda e, f: (e, 0, f)),
        ],
        out_specs=pl.BlockSpec((TOKENS, HIDDEN), lambda e, f: (0, 0)),
        out_shape=jax.ShapeDtypeStruct((TOKENS, HIDDEN), jnp.float32),
        compiler_params=pltpu.CompilerParams(
            dimension_semantics=("arbitrary", "arbitrary"),
        ),
    )(x, p, w1, w3, w2)
    return out


def kernel(hidden_states, gate_w, w1, w2, w3):
    b, s, h = hidden_states.shape
    x = hidden_states.reshape(-1, h)
    out = _moe(x, gate_w, w1, w2, w3)
    return out.reshape(b, s, h)


# speedup vs baseline: 1.7667x; 1.3887x over previous
import functools

import jax
import jax.numpy as jnp
from jax import lax
from jax.experimental import pallas as pl
from jax.experimental.pallas import tpu as pltpu
from jax.experimental.pallas import tpu_sc as plsc

NUM_EXPERTS = 8
HIDDEN = 2048
FFN = 5632
TOKENS = 2048
ASSIGN = 2 * TOKENS

B = 512
N_B = ASSIGN // B
S_ROWS = 8192
SAC = S_ROWS // B - 1
F_CHUNK = 512
N_F = FFN // F_CHUNK

NW = 32
TPW = TOKENS // NW
SC_CH = 32
CB_CH = 16
L = 16


def _router_body(x_ref, gate_ref, pos0_ref, pos1_ref, wv0_ref, wv1_ref,
                 bi_ref):
    x = x_ref[...]
    logits = lax.dot_general(x, gate_ref[...], (((1,), (1,)), ((), ())),
                             preferred_element_type=jnp.float32)
    e_iota = lax.broadcasted_iota(jnp.int32, (TOKENS, NUM_EXPERTS), 1)
    m1 = jnp.max(logits, axis=1, keepdims=True)
    i1 = jnp.min(jnp.where(logits >= m1, e_iota, NUM_EXPERTS), axis=1,
                 keepdims=True)
    oh1 = e_iota == i1
    logits2 = jnp.where(oh1, jnp.float32(-1e30), logits)
    m2 = jnp.max(logits2, axis=1, keepdims=True)
    i2 = jnp.min(jnp.where(logits2 >= m2, e_iota, NUM_EXPERTS), axis=1,
                 keepdims=True)
    oh2 = e_iota == i2
    w0 = 1.0 / (1.0 + jnp.exp(m2 - m1))
    w1 = 1.0 - w0
    c2 = oh1.astype(jnp.float32) + oh2.astype(jnp.float32)
    r_iota = lax.broadcasted_iota(jnp.int32, (TOKENS, TOKENS), 0)
    c_iota = lax.broadcasted_iota(jnp.int32, (TOKENS, TOKENS), 1)
    tri = (r_iota > c_iota).astype(jnp.float32)
    s2 = jnp.round(lax.dot_general(tri, c2, (((1,), (0,)), ((), ())),
                                   preferred_element_type=jnp.float32))
    counts = jnp.sum(c2, axis=0, keepdims=True)
    nb = jnp.floor((counts + (B - 1)) / B)
    e8r = lax.broadcasted_iota(jnp.int32, (NUM_EXPERTS, NUM_EXPERTS), 0)
    e8c = lax.broadcasted_iota(jnp.int32, (NUM_EXPERTS, NUM_EXPERTS), 1)
    tri8 = (e8r < e8c).astype(jnp.float32)
    brow = jnp.round(lax.dot_general(nb, tri8, (((1,), (0,)), ((), ()))))
    po = brow * B
    pos0 = jnp.sum(jnp.where(oh1, s2 + po, 0.0), axis=1, keepdims=True)
    pos1 = jnp.sum(jnp.where(oh2, s2 + po, 0.0), axis=1, keepdims=True)
    pos0_ref[...] = pos0.astype(jnp.int32)
    pos1_ref[...] = pos1.astype(jnp.int32)
    wv0_ref[...] = w0
    wv1_ref[...] = w1
    nb_b = jnp.broadcast_to(nb.astype(jnp.int32), (NUM_EXPERTS, NUM_EXPERTS))
    br_b = jnp.broadcast_to(brow.astype(jnp.int32),
                            (NUM_EXPERTS, NUM_EXPERTS))
    bi_ref[...] = jnp.where(e8r == 0, nb_b, jnp.where(e8r == 1, br_b, 0))


_MESH = plsc.VectorSubcoreMesh(core_axis_name="c", subcore_axis_name="s")


@functools.partial(
    pl.kernel,
    out_type=jax.ShapeDtypeStruct((S_ROWS, HIDDEN), jnp.float32),
    mesh=_MESH,
    scratch_types=[
        pltpu.VMEM((2, SC_CH), jnp.int32),
        pltpu.VMEM((SC_CH, HIDDEN), jnp.float32),
        pltpu.SemaphoreType.DMA,
        pltpu.SemaphoreType.DMA,
    ],
)
def _sc_scatter_x(pos0, pos1, x_hbm, xs_hbm, idx_v, buf, sem0, sem1):
    wid = lax.axis_index("s") * 2 + lax.axis_index("c")
    base = wid * TPW

    def body(c, _):
        tb = base + c * SC_CH
        pltpu.sync_copy(pos0.at[pl.ds(tb, SC_CH)], idx_v.at[0])
        pltpu.sync_copy(pos1.at[pl.ds(tb, SC_CH)], idx_v.at[1])
        pltpu.sync_copy(x_hbm.at[pl.ds(tb, SC_CH)], buf)
        c0 = pltpu.async_copy(buf, xs_hbm.at[idx_v.at[0]], sem0)
        c1 = pltpu.async_copy(buf, xs_hbm.at[idx_v.at[1]], sem1)
        c0.wait()
        c1.wait()
        return 0
    lax.fori_loop(0, TPW // SC_CH, body, 0)


def _group_body(brow_ref, nb_ref, x_ref, w1_ref, w3_ref, w2_ref, y_ref):
    b = pl.program_id(1)
    f = pl.program_id(2)
    active = b < nb_ref[pl.program_id(0)]

    @pl.when(active)
    def _():
        x = x_ref[...].astype(jnp.bfloat16)
        w1 = w1_ref[0].astype(jnp.bfloat16)
        w3 = w3_ref[0].astype(jnp.bfloat16)
        w2 = w2_ref[0].astype(jnp.bfloat16)
        t1 = lax.dot_general(x, w1, (((1,), (1,)), ((), ())),
                             preferred_element_type=jnp.float32)
        t3 = lax.dot_general(x, w3, (((1,), (1,)), ((), ())),
                             preferred_element_type=jnp.float32)
        h = (t1 * jax.nn.sigmoid(t1) * t3).astype(jnp.bfloat16)
        cur = lax.dot_general(h, w2, (((1,), (1,)), ((), ())),
                              preferred_element_type=jnp.float32)

        @pl.when(f == 0)
        def _():
            y_ref[...] = jnp.zeros_like(y_ref)

        y_ref[...] += cur


def _row_idx(e, b, brow, nb):
    return jnp.where(nb[e] > 0, brow[e] + jnp.minimum(b, nb[e] - 1), SAC)


@functools.partial(
    pl.kernel,
    out_type=jax.ShapeDtypeStruct((TOKENS, HIDDEN), jnp.float32),
    mesh=_MESH,
    scratch_types=[
        pltpu.VMEM((CB_CH,), jnp.int32),
        pltpu.VMEM((CB_CH,), jnp.int32),
        pltpu.VMEM((CB_CH,), jnp.float32),
        pltpu.VMEM((CB_CH,), jnp.float32),
        pltpu.VMEM((CB_CH, HIDDEN), jnp.float32),
        pltpu.VMEM((CB_CH, HIDDEN), jnp.float32),
        pltpu.SemaphoreType.DMA,
        pltpu.SemaphoreType.DMA,
    ],
)
def _sc_combine(pos0, pos1, wv0, wv1, y_hbm, out_hbm,
                i0_v, i1_v, w0_v, w1_v, b0, b1, sem0, sem1):
    wid = lax.axis_index("s") * 2 + lax.axis_index("c")
    base = wid * TPW

    def body(c, _):
        tb = base + c * CB_CH
        pltpu.sync_copy(pos0.at[pl.ds(tb, CB_CH)], i0_v)
        pltpu.sync_copy(pos1.at[pl.ds(tb, CB_CH)], i1_v)
        pltpu.sync_copy(wv0.at[pl.ds(tb, CB_CH)], w0_v)
        pltpu.sync_copy(wv1.at[pl.ds(tb, CB_CH)], w1_v)
        c0 = pltpu.async_copy(y_hbm.at[i0_v], b0, sem0)
        c1 = pltpu.async_copy(y_hbm.at[i1_v], b1, sem1)
        c0.wait()
        c1.wait()
        w0g = w0_v[pl.ds(0, CB_CH)]
        w1g = w1_v[pl.ds(0, CB_CH)]
        for j in range(CB_CH):
            a = w0g[j]
            bw = w1g[j]

            def col_body(kk, _2, j=j, a=a, bw=bw):
                b0[j, pl.ds(kk * L, L)] = (a * b0[j, pl.ds(kk * L, L)]
                                           + bw * b1[j, pl.ds(kk * L, L)])
                return 0
            lax.fori_loop(0, HIDDEN // L, col_body, 0)
        pltpu.sync_copy(b0, out_hbm.at[pl.ds(tb, CB_CH)])
        return 0
    lax.fori_loop(0, TPW // CB_CH, body, 0)


@jax.jit
def _moe(x, gate_w, w1, w2, w3):
    pos0, pos1, wv0, wv1, bi = pl.pallas_call(
        _router_body,
        out_shape=(
            jax.ShapeDtypeStruct((TOKENS, 1), jnp.int32),
            jax.ShapeDtypeStruct((TOKENS, 1), jnp.int32),
            jax.ShapeDtypeStruct((TOKENS, 1), jnp.float32),
            jax.ShapeDtypeStruct((TOKENS, 1), jnp.float32),
            jax.ShapeDtypeStruct((NUM_EXPERTS, NUM_EXPERTS), jnp.int32),
        ),
    )(x, gate_w)
    nb = bi[0]
    brow = bi[1]
    p0 = pos0.reshape(-1)
    p1 = pos1.reshape(-1)
    xs = _sc_scatter_x(p0, p1, x)

    grid_spec = pltpu.PrefetchScalarGridSpec(
        num_scalar_prefetch=2,
        grid=(NUM_EXPERTS, N_B, N_F),
        in_specs=[
            pl.BlockSpec((B, HIDDEN),
                         lambda e, b, f, brow, nb: (_row_idx(e, b, brow, nb),
                                                    0)),
            pl.BlockSpec((1, F_CHUNK, HIDDEN),
                         lambda e, b, f, brow, nb:
                         (e, jnp.where(b < nb[e], f, 0), 0)),
            pl.BlockSpec((1, F_CHUNK, HIDDEN),
                         lambda e, b, f, brow, nb:
                         (e, jnp.where(b < nb[e], f, 0), 0)),
            pl.BlockSpec((1, HIDDEN, F_CHUNK),
                         lambda e, b, f, brow, nb:
                         (e, 0, jnp.where(b < nb[e], f, 0))),
        ],
        out_specs=pl.BlockSpec(
            (B, HIDDEN),
            lambda e, b, f, brow, nb: (_row_idx(e, b, brow, nb), 0)),
    )
    y = pl.pallas_call(
        _group_body,
        grid_spec=grid_spec,
        out_shape=jax.ShapeDtypeStruct((S_ROWS, HIDDEN), jnp.float32),
        compiler_params=pltpu.CompilerParams(
            dimension_semantics=("arbitrary", "arbitrary", "arbitrary"),
        ),
    )(brow, nb, xs, w1, w3, w2)

    return _sc_combine(p0, p1, wv0.reshape(-1), wv1.reshape(-1), y)


def kernel(hidden_states, gate_w, w1, w2, w3):
    b, s, h = hidden_states.shape
    x = hidden_states.reshape(-1, h)
    out = _moe(x, gate_w, w1, w2, w3)
    return out.reshape(b, s, h)


# speedup vs baseline: 1.9671x; 1.1134x over previous
import functools

import jax
import jax.numpy as jnp
from jax import lax
from jax.experimental import pallas as pl
from jax.experimental.pallas import tpu as pltpu
from jax.experimental.pallas import tpu_sc as plsc

NUM_EXPERTS = 8
HIDDEN = 2048
FFN = 5632
TOKENS = 2048
ASSIGN = 2 * TOKENS

B = 512
N_TILES = 16
S_ROWS = N_TILES * B
F_CHUNK = 512
N_F = FFN // F_CHUNK

NW = 32
TPW = TOKENS // NW
SC_CH = 32
CB_CH = 16
L = 16


def _router_body(x_ref, gate_ref, pos0_ref, pos1_ref, wv0_ref, wv1_ref,
                 bi_ref):
    x = x_ref[...]
    logits = lax.dot_general(x, gate_ref[...], (((1,), (1,)), ((), ())),
                             preferred_element_type=jnp.float32)
    e_iota = lax.broadcasted_iota(jnp.int32, (TOKENS, NUM_EXPERTS), 1)
    m1 = jnp.max(logits, axis=1, keepdims=True)
    i1 = jnp.min(jnp.where(logits >= m1, e_iota, NUM_EXPERTS), axis=1,
                 keepdims=True)
    oh1 = e_iota == i1
    logits2 = jnp.where(oh1, jnp.float32(-1e30), logits)
    m2 = jnp.max(logits2, axis=1, keepdims=True)
    i2 = jnp.min(jnp.where(logits2 >= m2, e_iota, NUM_EXPERTS), axis=1,
                 keepdims=True)
    oh2 = e_iota == i2
    w0 = 1.0 / (1.0 + jnp.exp(m2 - m1))
    w1 = 1.0 - w0
    c2 = oh1.astype(jnp.float32) + oh2.astype(jnp.float32)
    r_iota = lax.broadcasted_iota(jnp.int32, (TOKENS, TOKENS), 0)
    c_iota = lax.broadcasted_iota(jnp.int32, (TOKENS, TOKENS), 1)
    tri = (r_iota > c_iota).astype(jnp.float32)
    s2 = jnp.round(lax.dot_general(tri, c2, (((1,), (0,)), ((), ())),
                                   preferred_element_type=jnp.float32))
    counts = jnp.sum(c2, axis=0, keepdims=True)
    nb = jnp.floor((counts + (B - 1)) / B)
    e8r = lax.broadcasted_iota(jnp.int32, (NUM_EXPERTS, NUM_EXPERTS), 0)
    e8c = lax.broadcasted_iota(jnp.int32, (NUM_EXPERTS, NUM_EXPERTS), 1)
    tri8 = (e8r < e8c).astype(jnp.float32)
    brow = jnp.round(lax.dot_general(nb, tri8, (((1,), (0,)), ((), ()))))
    po = brow * B
    pos0 = jnp.sum(jnp.where(oh1, s2 + po, 0.0), axis=1, keepdims=True)
    pos1 = jnp.sum(jnp.where(oh2, s2 + po, 0.0), axis=1, keepdims=True)
    pos0_ref[...] = pos0.astype(jnp.int32)
    pos1_ref[...] = pos1.astype(jnp.int32)
    wv0_ref[...] = w0
    wv1_ref[...] = w1
    g_iota = lax.broadcasted_iota(jnp.int32, (N_TILES, NUM_EXPERTS),
                                  0).astype(jnp.float32)
    brow_b = jnp.broadcast_to(brow, (N_TILES, NUM_EXPERTS))
    nb_b = jnp.broadcast_to(nb, (N_TILES, NUM_EXPERTS))
    cmp = (brow_b <= g_iota) & (g_iota < brow_b + nb_b)
    ef = lax.broadcasted_iota(jnp.int32, (N_TILES, NUM_EXPERTS),
                              1).astype(jnp.float32)
    eot = jnp.sum(jnp.where(cmp, ef, 0.0), axis=1, keepdims=True)
    val = jnp.sum(jnp.where(cmp, 1.0, 0.0), axis=1, keepdims=True)
    lane16 = lax.broadcasted_iota(jnp.int32, (N_TILES, NUM_EXPERTS), 1)
    eot_b = jnp.broadcast_to(eot, (N_TILES, NUM_EXPERTS)).astype(jnp.int32)
    val_b = jnp.broadcast_to(val, (N_TILES, NUM_EXPERTS)).astype(jnp.int32)
    bi_ref[...] = jnp.where(lane16 == 0, eot_b,
                            jnp.where(lane16 == 1, val_b, 0))


_MESH = plsc.VectorSubcoreMesh(core_axis_name="c", subcore_axis_name="s")


@functools.partial(
    pl.kernel,
    out_type=jax.ShapeDtypeStruct((S_ROWS, HIDDEN), jnp.float32),
    mesh=_MESH,
    scratch_types=[
        pltpu.VMEM((2, SC_CH), jnp.int32),
        pltpu.VMEM((SC_CH, HIDDEN), jnp.float32),
        pltpu.SemaphoreType.DMA,
        pltpu.SemaphoreType.DMA,
    ],
)
def _sc_scatter_x(pos0, pos1, x_hbm, xs_hbm, idx_v, buf, sem0, sem1):
    wid = lax.axis_index("s") * 2 + lax.axis_index("c")
    base = wid * TPW

    def body(c, _):
        tb = base + c * SC_CH
        pltpu.sync_copy(pos0.at[pl.ds(tb, SC_CH)], idx_v.at[0])
        pltpu.sync_copy(pos1.at[pl.ds(tb, SC_CH)], idx_v.at[1])
        pltpu.sync_copy(x_hbm.at[pl.ds(tb, SC_CH)], buf)
        c0 = pltpu.async_copy(buf, xs_hbm.at[idx_v.at[0]], sem0)
        c1 = pltpu.async_copy(buf, xs_hbm.at[idx_v.at[1]], sem1)
        c0.wait()
        c1.wait()
        return 0
    lax.fori_loop(0, TPW // SC_CH, body, 0)


def _group_body(eot_ref, val_ref, x_ref, w1_ref, w3_ref, w2_ref, y_ref):
    f = pl.program_id(1)
    active = val_ref[pl.program_id(0)] > 0

    @pl.when(active)
    def _():
        x = x_ref[...].astype(jnp.bfloat16)
        w1 = w1_ref[0].astype(jnp.bfloat16)
        w3 = w3_ref[0].astype(jnp.bfloat16)
        w2 = w2_ref[0].astype(jnp.bfloat16)
        t1 = lax.dot_general(x, w1, (((1,), (1,)), ((), ())),
                             preferred_element_type=jnp.float32)
        t3 = lax.dot_general(x, w3, (((1,), (1,)), ((), ())),
                             preferred_element_type=jnp.float32)
        h = (t1 * jax.nn.sigmoid(t1) * t3).astype(jnp.bfloat16)
        cur = lax.dot_general(h, w2, (((1,), (1,)), ((), ())),
                              preferred_element_type=jnp.float32)

        @pl.when(f == 0)
        def _():
            y_ref[...] = jnp.zeros_like(y_ref)

        y_ref[...] += cur


@functools.partial(
    pl.kernel,
    out_type=jax.ShapeDtypeStruct((TOKENS, HIDDEN), jnp.float32),
    mesh=_MESH,
    scratch_types=[
        pltpu.VMEM((CB_CH,), jnp.int32),
        pltpu.VMEM((CB_CH,), jnp.int32),
        pltpu.VMEM((CB_CH,), jnp.float32),
        pltpu.VMEM((CB_CH,), jnp.float32),
        pltpu.VMEM((CB_CH, HIDDEN), jnp.float32),
        pltpu.VMEM((CB_CH, HIDDEN), jnp.float32),
        pltpu.SemaphoreType.DMA,
        pltpu.SemaphoreType.DMA,
    ],
)
def _sc_combine(pos0, pos1, wv0, wv1, y_hbm, out_hbm,
                i0_v, i1_v, w0_v, w1_v, b0, b1, sem0, sem1):
    wid = lax.axis_index("s") * 2 + lax.axis_index("c")
    base = wid * TPW

    def body(c, _):
        tb = base + c * CB_CH
        pltpu.sync_copy(pos0.at[pl.ds(tb, CB_CH)], i0_v)
        pltpu.sync_copy(pos1.at[pl.ds(tb, CB_CH)], i1_v)
        pltpu.sync_copy(wv0.at[pl.ds(tb, CB_CH)], w0_v)
        pltpu.sync_copy(wv1.at[pl.ds(tb, CB_CH)], w1_v)
        c0 = pltpu.async_copy(y_hbm.at[i0_v], b0, sem0)
        c1 = pltpu.async_copy(y_hbm.at[i1_v], b1, sem1)
        c0.wait()
        c1.wait()
        w0g = w0_v[pl.ds(0, CB_CH)]
        w1g = w1_v[pl.ds(0, CB_CH)]
        for j in range(CB_CH):
            a = w0g[j]
            bw = w1g[j]

            def col_body(kk, _2, j=j, a=a, bw=bw):
                b0[j, pl.ds(kk * L, L)] = (a * b0[j, pl.ds(kk * L, L)]
                                           + bw * b1[j, pl.ds(kk * L, L)])
                return 0
            lax.fori_loop(0, HIDDEN // L, col_body, 0)
        pltpu.sync_copy(b0, out_hbm.at[pl.ds(tb, CB_CH)])
        return 0
    lax.fori_loop(0, TPW // CB_CH, body, 0)


@jax.jit
def _moe(x, gate_w, w1, w2, w3):
    pos0, pos1, wv0, wv1, bi = pl.pallas_call(
        _router_body,
        out_shape=(
            jax.ShapeDtypeStruct((TOKENS, 1), jnp.int32),
            jax.ShapeDtypeStruct((TOKENS, 1), jnp.int32),
            jax.ShapeDtypeStruct((TOKENS, 1), jnp.float32),
            jax.ShapeDtypeStruct((TOKENS, 1), jnp.float32),
            jax.ShapeDtypeStruct((N_TILES, NUM_EXPERTS), jnp.int32),
        ),
    )(x, gate_w)
    eot = bi[:, 0]
    val = bi[:, 1]
    p0 = pos0.reshape(-1)
    p1 = pos1.reshape(-1)
    xs = _sc_scatter_x(p0, p1, x)

    grid_spec = pltpu.PrefetchScalarGridSpec(
        num_scalar_prefetch=2,
        grid=(N_TILES, N_F),
        in_specs=[
            pl.BlockSpec((B, HIDDEN), lambda g, f, eot, val: (g, 0)),
            pl.BlockSpec((1, F_CHUNK, HIDDEN),
                         lambda g, f, eot, val:
                         (eot[g], jnp.where(val[g] > 0, f, 0), 0)),
            pl.BlockSpec((1, F_CHUNK, HIDDEN),
                         lambda g, f, eot, val:
                         (eot[g], jnp.where(val[g] > 0, f, 0), 0)),
            pl.BlockSpec((1, HIDDEN, F_CHUNK),
                         lambda g, f, eot, val:
                         (eot[g], 0, jnp.where(val[g] > 0, f, 0))),
        ],
        out_specs=pl.BlockSpec((B, HIDDEN), lambda g, f, eot, val: (g, 0)),
    )
    y = pl.pallas_call(
        _group_body,
        grid_spec=grid_spec,
        out_shape=jax.ShapeDtypeStruct((S_ROWS, HIDDEN), jnp.float32),
        compiler_params=pltpu.CompilerParams(
            dimension_semantics=("arbitrary", "arbitrary"),
        ),
    )(eot, val, xs, w1, w3, w2)

    return _sc_combine(p0, p1, wv0.reshape(-1), wv1.reshape(-1), y)


def kernel(hidden_states, gate_w, w1, w2, w3):
    b, s, h = hidden_states.shape
    x = hidden_states.reshape(-1, h)
    out = _moe(x, gate_w, w1, w2, w3)
    return out.reshape(b, s, h)


# speedup vs baseline: 2.7615x; 1.4038x over previous
import functools

import jax
import jax.numpy as jnp
from jax import lax
from jax.experimental import pallas as pl
from jax.experimental.pallas import tpu as pltpu
from jax.experimental.pallas import tpu_sc as plsc

NUM_EXPERTS = 8
HIDDEN = 2048
FFN = 5632
TOKENS = 2048
ASSIGN = 2 * TOKENS

B = 576
N_TILES = 15
S_ROWS = N_TILES * B
F_CHUNK = 512
N_F = FFN // F_CHUNK

NW = 32
TPW = TOKENS // NW
SC_CH = 32
CB_CH = 16
L = 16


def _router_body(x_ref, gate_ref, pos0_ref, pos1_ref, wv0_ref, wv1_ref,
                 bi_ref):
    x = x_ref[...]
    logits = lax.dot_general(x, gate_ref[...], (((1,), (1,)), ((), ())),
                             preferred_element_type=jnp.float32)
    e_iota = lax.broadcasted_iota(jnp.int32, (TOKENS, NUM_EXPERTS), 1)
    m1 = jnp.max(logits, axis=1, keepdims=True)
    i1 = jnp.min(jnp.where(logits >= m1, e_iota, NUM_EXPERTS), axis=1,
                 keepdims=True)
    oh1 = e_iota == i1
    logits2 = jnp.where(oh1, jnp.float32(-1e30), logits)
    m2 = jnp.max(logits2, axis=1, keepdims=True)
    i2 = jnp.min(jnp.where(logits2 >= m2, e_iota, NUM_EXPERTS), axis=1,
                 keepdims=True)
    oh2 = e_iota == i2
    w0 = 1.0 / (1.0 + jnp.exp(m2 - m1))
    w1 = 1.0 - w0
    c2 = oh1.astype(jnp.float32) + oh2.astype(jnp.float32)
    r_iota = lax.broadcasted_iota(jnp.int32, (TOKENS, TOKENS), 0)
    c_iota = lax.broadcasted_iota(jnp.int32, (TOKENS, TOKENS), 1)
    tri = (r_iota > c_iota).astype(jnp.float32)
    s2 = jnp.round(lax.dot_general(tri, c2, (((1,), (0,)), ((), ())),
                                   preferred_element_type=jnp.float32))
    counts = jnp.sum(c2, axis=0, keepdims=True)
    nb = jnp.floor((counts + (B - 1)) / B)
    e8r = lax.broadcasted_iota(jnp.int32, (NUM_EXPERTS, NUM_EXPERTS), 0)
    e8c = lax.broadcasted_iota(jnp.int32, (NUM_EXPERTS, NUM_EXPERTS), 1)
    tri8 = (e8r < e8c).astype(jnp.float32)
    brow = jnp.round(lax.dot_general(nb, tri8, (((1,), (0,)), ((), ()))))
    po = brow * B
    pos0 = jnp.sum(jnp.where(oh1, s2 + po, 0.0), axis=1, keepdims=True)
    pos1 = jnp.sum(jnp.where(oh2, s2 + po, 0.0), axis=1, keepdims=True)
    pos0_ref[...] = pos0.astype(jnp.int32)
    pos1_ref[...] = pos1.astype(jnp.int32)
    wv0_ref[...] = w0
    wv1_ref[...] = w1
    g_iota = lax.broadcasted_iota(jnp.int32, (N_TILES, NUM_EXPERTS),
                                  0).astype(jnp.float32)
    brow_b = jnp.broadcast_to(brow, (N_TILES, NUM_EXPERTS))
    nb_b = jnp.broadcast_to(nb, (N_TILES, NUM_EXPERTS))
    cmp = (brow_b <= g_iota) & (g_iota < brow_b + nb_b)
    ef = lax.broadcasted_iota(jnp.int32, (N_TILES, NUM_EXPERTS),
                              1).astype(jnp.float32)
    eot = jnp.sum(jnp.where(cmp, ef, 0.0), axis=1, keepdims=True)
    val = jnp.sum(jnp.where(cmp, 1.0, 0.0), axis=1, keepdims=True)
    lane16 = lax.broadcasted_iota(jnp.int32, (N_TILES, NUM_EXPERTS), 1)
    eot_b = jnp.broadcast_to(eot, (N_TILES, NUM_EXPERTS)).astype(jnp.int32)
    val_b = jnp.broadcast_to(val, (N_TILES, NUM_EXPERTS)).astype(jnp.int32)
    bi_ref[...] = jnp.where(lane16 == 0, eot_b,
                            jnp.where(lane16 == 1, val_b, 0))


_MESH = plsc.VectorSubcoreMesh(core_axis_name="c", subcore_axis_name="s")


@functools.partial(
    pl.kernel,
    out_type=jax.ShapeDtypeStruct((S_ROWS, HIDDEN), jnp.float32),
    mesh=_MESH,
    scratch_types=[
        pltpu.VMEM((2, SC_CH), jnp.int32),
        pltpu.VMEM((SC_CH, HIDDEN), jnp.float32),
        pltpu.SemaphoreType.DMA,
        pltpu.SemaphoreType.DMA,
    ],
)
def _sc_scatter_x(pos0, pos1, x_hbm, xs_hbm, idx_v, buf, sem0, sem1):
    wid = lax.axis_index("s") * 2 + lax.axis_index("c")
    base = wid * TPW

    def body(c, _):
        tb = base + c * SC_CH
        pltpu.sync_copy(pos0.at[pl.ds(tb, SC_CH)], idx_v.at[0])
        pltpu.sync_copy(pos1.at[pl.ds(tb, SC_CH)], idx_v.at[1])
        pltpu.sync_copy(x_hbm.at[pl.ds(tb, SC_CH)], buf)
        c0 = pltpu.async_copy(buf, xs_hbm.at[idx_v.at[0]], sem0)
        c1 = pltpu.async_copy(buf, xs_hbm.at[idx_v.at[1]], sem1)
        c0.wait()
        c1.wait()
        return 0
    lax.fori_loop(0, TPW // SC_CH, body, 0)


def _group_body(eot_ref, val_ref, x_ref, w1_ref, w3_ref, w2_ref, y_ref):
    f = pl.program_id(1)
    active = val_ref[pl.program_id(0)] > 0

    @pl.when(active)
    def _():
        x = x_ref[...].astype(jnp.bfloat16)
        w1 = w1_ref[0].astype(jnp.bfloat16)
        w3 = w3_ref[0].astype(jnp.bfloat16)
        w2 = w2_ref[0].astype(jnp.bfloat16)
        t1 = lax.dot_general(x, w1, (((1,), (1,)), ((), ())),
                             preferred_element_type=jnp.float32)
        t3 = lax.dot_general(x, w3, (((1,), (1,)), ((), ())),
                             preferred_element_type=jnp.float32)
        h = (t1 * jax.nn.sigmoid(t1) * t3).astype(jnp.bfloat16)
        cur = lax.dot_general(h, w2, (((1,), (1,)), ((), ())),
                              preferred_element_type=jnp.float32)

        @pl.when(f == 0)
        def _():
            y_ref[...] = jnp.zeros_like(y_ref)

        y_ref[...] += cur


@functools.partial(
    pl.kernel,
    out_type=jax.ShapeDtypeStruct((TOKENS, HIDDEN), jnp.float32),
    mesh=_MESH,
    scratch_types=[
        pltpu.VMEM((CB_CH,), jnp.int32),
        pltpu.VMEM((CB_CH,), jnp.int32),
        pltpu.VMEM((CB_CH,), jnp.float32),
        pltpu.VMEM((CB_CH,), jnp.float32),
        pltpu.VMEM((CB_CH, HIDDEN), jnp.float32),
        pltpu.VMEM((CB_CH, HIDDEN), jnp.float32),
        pltpu.SemaphoreType.DMA,
        pltpu.SemaphoreType.DMA,
    ],
)
def _sc_combine(pos0, pos1, wv0, wv1, y_hbm, out_hbm,
                i0_v, i1_v, w0_v, w1_v, b0, b1, sem0, sem1):
    wid = lax.axis_index("s") * 2 + lax.axis_index("c")
    base = wid * TPW

    def body(c, _):
        tb = base + c * CB_CH
        pltpu.sync_copy(pos0.at[pl.ds(tb, CB_CH)], i0_v)
        pltpu.sync_copy(pos1.at[pl.ds(tb, CB_CH)], i1_v)
        pltpu.sync_copy(wv0.at[pl.ds(tb, CB_CH)], w0_v)
        pltpu.sync_copy(wv1.at[pl.ds(tb, CB_CH)], w1_v)
        c0 = pltpu.async_copy(y_hbm.at[i0_v], b0, sem0)
        c1 = pltpu.async_copy(y_hbm.at[i1_v], b1, sem1)
        c0.wait()
        c1.wait()
        w0g = w0_v[pl.ds(0, CB_CH)]
        w1g = w1_v[pl.ds(0, CB_CH)]
        for j in range(CB_CH):
            a = w0g[j]
            bw = w1g[j]

            def col_body(kk, _2, j=j, a=a, bw=bw):
                b0[j, pl.ds(kk * L, L)] = (a * b0[j, pl.ds(kk * L, L)]
                                           + bw * b1[j, pl.ds(kk * L, L)])
                return 0
            lax.fori_loop(0, HIDDEN // L, col_body, 0)
        pltpu.sync_copy(b0, out_hbm.at[pl.ds(tb, CB_CH)])
        return 0
    lax.fori_loop(0, TPW // CB_CH, body, 0)


@jax.jit
def _moe(x, gate_w, w1, w2, w3):
    pos0, pos1, wv0, wv1, bi = pl.pallas_call(
        _router_body,
        out_shape=(
            jax.ShapeDtypeStruct((TOKENS, 1), jnp.int32),
            jax.ShapeDtypeStruct((TOKENS, 1), jnp.int32),
            jax.ShapeDtypeStruct((TOKENS, 1), jnp.float32),
            jax.ShapeDtypeStruct((TOKENS, 1), jnp.float32),
            jax.ShapeDtypeStruct((N_TILES, NUM_EXPERTS), jnp.int32),
        ),
    )(x, gate_w)
    eot = bi[:, 0]
    val = bi[:, 1]
    p0 = pos0.reshape(-1)
    p1 = pos1.reshape(-1)
    xs = _sc_scatter_x(p0, p1, x)

    grid_spec = pltpu.PrefetchScalarGridSpec(
        num_scalar_prefetch=2,
        grid=(N_TILES, N_F),
        in_specs=[
            pl.BlockSpec((B, HIDDEN),
                         lambda g, f, eot, val:
                         (jnp.where(val[g] > 0, g, 0), 0)),
            pl.BlockSpec((1, F_CHUNK, HIDDEN),
                         lambda g, f, eot, val:
                         (eot[g], jnp.where(val[g] > 0, f, 0), 0)),
            pl.BlockSpec((1, F_CHUNK, HIDDEN),
                         lambda g, f, eot, val:
                         (eot[g], jnp.where(val[g] > 0, f, 0), 0)),
            pl.BlockSpec((1, HIDDEN, F_CHUNK),
                         lambda g, f, eot, val:
                         (eot[g], 0, jnp.where(val[g] > 0, f, 0))),
        ],
        out_specs=pl.BlockSpec((B, HIDDEN), lambda g, f, eot, val: (g, 0)),
    )
    y = pl.pallas_call(
        _group_body,
        grid_spec=grid_spec,
        out_shape=jax.ShapeDtypeStruct((S_ROWS, HIDDEN), jnp.float32),
        compiler_params=pltpu.CompilerParams(
            dimension_semantics=("arbitrary", "arbitrary"),
        ),
    )(eot, val, xs, w1, w3, w2)

    return _sc_combine(p0, p1, wv0.reshape(-1), wv1.reshape(-1), y)


def kernel(hidden_states, gate_w, w1, w2, w3):
    b, s, h = hidden_states.shape
    x = hidden_states.reshape(-1, h)
    out = _moe(x, gate_w, w1, w2, w3)
    return out.reshape(b, s, h)


# speedup vs baseline: 2.8442x; 1.0300x over previous
import functools

import jax
import jax.numpy as jnp
from jax import lax
from jax.experimental import pallas as pl
from jax.experimental.pallas import tpu as pltpu
from jax.experimental.pallas import tpu_sc as plsc

NUM_EXPERTS = 8
HIDDEN = 2048
FFN = 5632
TOKENS = 2048
ASSIGN = 2 * TOKENS

B = 576
N_TILES = 15
S_ROWS = N_TILES * B
F_CHUNK = 512
N_F = FFN // F_CHUNK

NW = 32
TPW = TOKENS // NW
SC_CH = 32
CB_CH = 16
L = 16


def _router_body(x_ref, gate_ref, pos0_ref, pos1_ref, wv0_ref, wv1_ref,
                 bi_ref):
    x = x_ref[...]
    logits = lax.dot_general(x, gate_ref[...], (((1,), (1,)), ((), ())),
                             preferred_element_type=jnp.float32)
    e_iota = lax.broadcasted_iota(jnp.int32, (TOKENS, NUM_EXPERTS), 1)
    m1 = jnp.max(logits, axis=1, keepdims=True)
    i1 = jnp.min(jnp.where(logits >= m1, e_iota, NUM_EXPERTS), axis=1,
                 keepdims=True)
    oh1 = e_iota == i1
    logits2 = jnp.where(oh1, jnp.float32(-1e30), logits)
    m2 = jnp.max(logits2, axis=1, keepdims=True)
    i2 = jnp.min(jnp.where(logits2 >= m2, e_iota, NUM_EXPERTS), axis=1,
                 keepdims=True)
    oh2 = e_iota == i2
    w0 = 1.0 / (1.0 + jnp.exp(m2 - m1))
    w1 = 1.0 - w0
    c2 = oh1.astype(jnp.float32) + oh2.astype(jnp.float32)
    r_iota = lax.broadcasted_iota(jnp.int32, (TOKENS, TOKENS), 0)
    c_iota = lax.broadcasted_iota(jnp.int32, (TOKENS, TOKENS), 1)
    tri = (r_iota > c_iota).astype(jnp.float32)
    s2 = jnp.round(lax.dot_general(tri, c2, (((1,), (0,)), ((), ())),
                                   preferred_element_type=jnp.float32))
    counts = jnp.sum(c2, axis=0, keepdims=True)
    nb = jnp.floor((counts + (B - 1)) / B)
    e8r = lax.broadcasted_iota(jnp.int32, (NUM_EXPERTS, NUM_EXPERTS), 0)
    e8c = lax.broadcasted_iota(jnp.int32, (NUM_EXPERTS, NUM_EXPERTS), 1)
    tri8 = (e8r < e8c).astype(jnp.float32)
    brow = jnp.round(lax.dot_general(nb, tri8, (((1,), (0,)), ((), ()))))
    po = brow * B
    pos0 = jnp.sum(jnp.where(oh1, s2 + po, 0.0), axis=1, keepdims=True)
    pos1 = jnp.sum(jnp.where(oh2, s2 + po, 0.0), axis=1, keepdims=True)
    pos0_ref[...] = pos0.astype(jnp.int32)
    pos1_ref[...] = pos1.astype(jnp.int32)
    wv0_ref[...] = w0
    wv1_ref[...] = w1
    g_iota = lax.broadcasted_iota(jnp.int32, (N_TILES, NUM_EXPERTS),
                                  0).astype(jnp.float32)
    brow_b = jnp.broadcast_to(brow, (N_TILES, NUM_EXPERTS))
    nb_b = jnp.broadcast_to(nb, (N_TILES, NUM_EXPERTS))
    cmp = (brow_b <= g_iota) & (g_iota < brow_b + nb_b)
    ef = lax.broadcasted_iota(jnp.int32, (N_TILES, NUM_EXPERTS),
                              1).astype(jnp.float32)
    eot = jnp.sum(jnp.where(cmp, ef, 0.0), axis=1, keepdims=True)
    val = jnp.sum(jnp.where(cmp, 1.0, 0.0), axis=1, keepdims=True)
    lane16 = lax.broadcasted_iota(jnp.int32, (N_TILES, NUM_EXPERTS), 1)
    eot_b = jnp.broadcast_to(eot, (N_TILES, NUM_EXPERTS)).astype(jnp.int32)
    val_b = jnp.broadcast_to(val, (N_TILES, NUM_EXPERTS)).astype(jnp.int32)
    bi_ref[...] = jnp.where(lane16 == 0, eot_b,
                            jnp.where(lane16 == 1, val_b, 0))


_MESH = plsc.VectorSubcoreMesh(core_axis_name="c", subcore_axis_name="s")


@functools.partial(
    pl.kernel,
    out_type=(jax.ShapeDtypeStruct((S_ROWS, HIDDEN), jnp.float32),
              jax.ShapeDtypeStruct((S_ROWS,), jnp.float32)),
    mesh=_MESH,
    scratch_types=[
        pltpu.VMEM((2, SC_CH), jnp.int32),
        pltpu.VMEM((SC_CH, HIDDEN), jnp.float32),
        pltpu.VMEM((2, SC_CH), jnp.float32),
        pltpu.SemaphoreType.DMA,
        pltpu.SemaphoreType.DMA,
        pltpu.SemaphoreType.DMA,
        pltpu.SemaphoreType.DMA,
    ],
)
def _sc_scatter_x(pos0, pos1, wv0, wv1, x_hbm, xs_hbm, wt_hbm,
                  idx_v, buf, wbuf, sem0, sem1, sem2, sem3):
    wid = lax.axis_index("s") * 2 + lax.axis_index("c")
    base = wid * TPW

    def body(c, _):
        tb = base + c * SC_CH
        pltpu.sync_copy(pos0.at[pl.ds(tb, SC_CH)], idx_v.at[0])
        pltpu.sync_copy(pos1.at[pl.ds(tb, SC_CH)], idx_v.at[1])
        pltpu.sync_copy(x_hbm.at[pl.ds(tb, SC_CH)], buf)
        pltpu.sync_copy(wv0.at[pl.ds(tb, SC_CH)], wbuf.at[0])
        pltpu.sync_copy(wv1.at[pl.ds(tb, SC_CH)], wbuf.at[1])
        c0 = pltpu.async_copy(buf, xs_hbm.at[idx_v.at[0]], sem0)
        c1 = pltpu.async_copy(buf, xs_hbm.at[idx_v.at[1]], sem1)
        c2 = pltpu.async_copy(wbuf.at[0], wt_hbm.at[idx_v.at[0]], sem2)
        c3 = pltpu.async_copy(wbuf.at[1], wt_hbm.at[idx_v.at[1]], sem3)
        c0.wait()
        c1.wait()
        c2.wait()
        c3.wait()
        return 0
    lax.fori_loop(0, TPW // SC_CH, body, 0)


def _group_body(eot_ref, val_ref, x_ref, w1_ref, w3_ref, w2_ref, wt_ref,
                y_ref):
    f = pl.program_id(1)
    active = val_ref[pl.program_id(0)] > 0

    @pl.when(active)
    def _():
        x = x_ref[...].astype(jnp.bfloat16)
        w1 = w1_ref[0].astype(jnp.bfloat16)
        w3 = w3_ref[0].astype(jnp.bfloat16)
        w2 = w2_ref[0].astype(jnp.bfloat16)
        t1 = lax.dot_general(x, w1, (((1,), (1,)), ((), ())),
                             preferred_element_type=jnp.float32)
        t3 = lax.dot_general(x, w3, (((1,), (1,)), ((), ())),
                             preferred_element_type=jnp.float32)
        h = (t1 * jax.nn.sigmoid(t1) * t3).astype(jnp.bfloat16)
        cur = lax.dot_general(h, w2, (((1,), (1,)), ((), ())),
                              preferred_element_type=jnp.float32)

        @pl.when(f == 0)
        def _():
            y_ref[...] = cur

        @pl.when(f > 0)
        def _():
            y_ref[...] += cur

        @pl.when(f == N_F - 1)
        def _():
            y_ref[...] *= wt_ref[...]


N_CB = TPW // CB_CH


@functools.partial(
    pl.kernel,
    out_type=jax.ShapeDtypeStruct((TOKENS, HIDDEN), jnp.float32),
    mesh=_MESH,
    scratch_types=[
        pltpu.VMEM((TPW,), jnp.int32),
        pltpu.VMEM((TPW,), jnp.int32),
        pltpu.VMEM((2, CB_CH, HIDDEN), jnp.float32),
        pltpu.SemaphoreType.DMA,
        pltpu.SemaphoreType.DMA,
        pltpu.SemaphoreType.DMA,
        pltpu.SemaphoreType.DMA,
    ],
)
def _sc_combine(pos0, pos1, y_hbm, out_hbm, i0_v, i1_v, bb, s0a, s0b,
                s1a, s1b):
    wid = lax.axis_index("s") * 2 + lax.axis_index("c")
    base = wid * TPW
    pltpu.sync_copy(pos0.at[pl.ds(base, TPW)], i0_v)
    pltpu.sync_copy(pos1.at[pl.ds(base, TPW)], i1_v)
    sems = ((s0a, s0b), (s1a, s1b))

    def fire(c, k):
        buf = bb.at[k]
        d0 = pltpu.async_copy(y_hbm.at[i0_v.at[pl.ds(c * CB_CH, CB_CH)]],
                              buf, sems[k][0])
        d1 = pltpu.async_copy(y_hbm.at[i1_v.at[pl.ds(c * CB_CH, CB_CH)]],
                              buf, sems[k][1], add=True)
        return d0, d1

    pend = fire(0, 0)
    for c in range(N_CB):
        nxt = fire(c + 1, (c + 1) % 2) if c + 1 < N_CB else None
        pend[0].wait()
        pend[1].wait()
        pltpu.sync_copy(bb.at[c % 2],
                        out_hbm.at[pl.ds(base + c * CB_CH, CB_CH)])
        pend = nxt


@jax.jit
def _moe(x, gate_w, w1, w2, w3):
    pos0, pos1, wv0, wv1, bi = pl.pallas_call(
        _router_body,
        out_shape=(
            jax.ShapeDtypeStruct((TOKENS, 1), jnp.int32),
            jax.ShapeDtypeStruct((TOKENS, 1), jnp.int32),
            jax.ShapeDtypeStruct((TOKENS, 1), jnp.float32),
            jax.ShapeDtypeStruct((TOKENS, 1), jnp.float32),
            jax.ShapeDtypeStruct((N_TILES, NUM_EXPERTS), jnp.int32),
        ),
    )(x, gate_w)
    eot = bi[:, 0]
    val = bi[:, 1]
    p0 = pos0.reshape(-1)
    p1 = pos1.reshape(-1)
    xs, wt = _sc_scatter_x(p0, p1, wv0.reshape(-1), wv1.reshape(-1), x)

    grid_spec = pltpu.PrefetchScalarGridSpec(
        num_scalar_prefetch=2,
        grid=(N_TILES, N_F),
        in_specs=[
            pl.BlockSpec((B, HIDDEN),
                         lambda g, f, eot, val:
                         (jnp.where(val[g] > 0, g, 0), 0)),
            pl.BlockSpec((1, F_CHUNK, HIDDEN),
                         lambda g, f, eot, val:
                         (eot[g], jnp.where(val[g] > 0, f, 0), 0)),
            pl.BlockSpec((1, F_CHUNK, HIDDEN),
                         lambda g, f, eot, val:
                         (eot[g], jnp.where(val[g] > 0, f, 0), 0)),
            pl.BlockSpec((1, HIDDEN, F_CHUNK),
                         lambda g, f, eot, val:
                         (eot[g], 0, jnp.where(val[g] > 0, f, 0))),
            pl.BlockSpec((B, 1), lambda g, f, eot, val:
                         (jnp.where(val[g] > 0, g, 0), 0)),
        ],
        out_specs=pl.BlockSpec((B, HIDDEN), lambda g, f, eot, val: (g, 0)),
    )
    y = pl.pallas_call(
        _group_body,
        grid_spec=grid_spec,
        out_shape=jax.ShapeDtypeStruct((S_ROWS, HIDDEN), jnp.float32),
        compiler_params=pltpu.CompilerParams(
            dimension_semantics=("arbitrary", "arbitrary"),
        ),
    )(eot, val, xs, w1, w3, w2, wt.reshape(S_ROWS, 1))

    return _sc_combine(p0, p1, y)


def kernel(hidden_states, gate_w, w1, w2, w3):
    b, s, h = hidden_states.shape
    x = hidden_states.reshape(-1, h)
    out = _moe(x, gate_w, w1, w2, w3)
    return out.reshape(b, s, h)
